# Initial kernel scaffold; baseline (speedup 1.0000x reference)
import jax
import jax.numpy as jnp
from jax.experimental import pallas as pl


def kernel(x, edge_index, eps, W0, W1, Wm1, bm1, Wm2, bm2, Wl1, bl1, Wl2, bl2):
    raise NotImplementedError("write your pallas kernel here")


# speedup vs baseline: 5.1834x; 5.1834x over previous
import functools

import jax
import jax.numpy as jnp
from jax import lax
from jax.experimental import pallas as pl
from jax.experimental.pallas import tpu as pltpu
from jax.experimental.pallas import tpu_sc as plsc

N = 10000
E = 320000
D_IN = 128
H = 64
OUT = 32

NW = 32
LN = 128
GS = 8
NG = 10
CH = NG * GS
EPW = CH * LN
E_PAD = NW * EPW
N_PAD = 10112
RPT = N_PAD // 16

_mesh = plsc.VectorSubcoreMesh(core_axis_name="c", subcore_axis_name="s")


@functools.partial(
    pl.kernel,
    mesh=_mesh,
    compiler_params=pltpu.CompilerParams(use_tc_tiling_on_sc=False),
    out_type=jax.ShapeDtypeStruct((2, N_PAD, H), jnp.float32),
    scratch_types=[
        pltpu.VMEM((CH, LN), jnp.int32),
        pltpu.VMEM((CH, LN), jnp.int32),
        pltpu.VMEM((GS, LN, H), jnp.float32),
        pltpu.VMEM_SHARED((N_PAD, H), jnp.float32),
        pltpu.SemaphoreType.DMA((GS,)),
        pltpu.SemaphoreType.DMA,
    ],
)
def _segsum_sc(rows_hbm, src_hbm, dst_hbm, zeros_hbm, out_hbm,
               src_v, dst_v, buf_v, acc_sh, gsem, ssem):
    c = lax.axis_index("c")
    s = lax.axis_index("s")
    wid = s * 2 + c

    pltpu.sync_copy(src_hbm.at[wid], src_v)
    pltpu.sync_copy(dst_hbm.at[wid], dst_v)
    pltpu.sync_copy(zeros_hbm.at[pl.ds(s * RPT, RPT)],
                    acc_sh.at[pl.ds(s * RPT, RPT)])
    plsc.subcore_barrier()

    def group(g, carry):
        j0 = g * GS
        gathers = [
            pltpu.async_copy(rows_hbm.at[src_v.at[j0 + b]], buf_v.at[b],
                             gsem.at[b])
            for b in range(GS)
        ]
        scatters = []
        for b in range(GS):
            gathers[b].wait()
            scatters.append(
                pltpu.async_copy(buf_v.at[b], acc_sh.at[dst_v.at[j0 + b]],
                                 ssem, add=True))
        for sc in scatters:
            sc.wait()
        return carry

    lax.fori_loop(0, NG, group, 0)
    plsc.subcore_barrier()

    pltpu.sync_copy(acc_sh.at[pl.ds(s * RPT, RPT)],
                    out_hbm.at[c, pl.ds(s * RPT, RPT)])


def _mm_body(x_ref, w_ref, o_ref):
    o_ref[...] = jnp.dot(x_ref[...], w_ref[...],
                         preferred_element_type=jnp.float32)


def _mid_body(a_ref, b_ref, w_ref, o_ref):
    m = a_ref[...] + b_ref[...]
    n = jnp.sqrt(jnp.sum(m * m, axis=1, keepdims=True))
    h = jnp.maximum(m / jnp.maximum(n, 1e-12), 0.0)
    o_ref[...] = jnp.dot(h, w_ref[...], preferred_element_type=jnp.float32)


def _head_body(a_ref, b_ref, eps_ref, wm1_ref, bm1_ref, wm2_ref, bm2_ref,
               wl1_ref, bl1_ref, wl2_ref, bl2_ref, o_ref):
    m = a_ref[...] + b_ref[...]
    n = jnp.sqrt(jnp.sum(m * m, axis=1, keepdims=True))
    h = jnp.maximum(m / jnp.maximum(n, 1e-12), 0.0)
    t1 = jnp.tanh(jnp.dot(h, wm1_ref[...], preferred_element_type=jnp.float32)
                  + bm1_ref[...])
    mu = jnp.dot(t1, wm2_ref[...], preferred_element_type=jnp.float32) \
        + bm2_ref[...]
    t2 = jnp.tanh(jnp.dot(h, wl1_ref[...], preferred_element_type=jnp.float32)
                  + bl1_ref[...])
    ls = jnp.dot(t2, wl2_ref[...], preferred_element_type=jnp.float32) \
        + bl2_ref[...]
    ls = jnp.minimum(ls, 10.0)
    o_ref[...] = mu + eps_ref[...] * jnp.exp(ls)


_BM = 1000


def _tc_matmul(x, w):
    return pl.pallas_call(
        _mm_body,
        grid=(N // _BM,),
        in_specs=[
            pl.BlockSpec((_BM, x.shape[1]), lambda i: (i, 0)),
            pl.BlockSpec(w.shape, lambda i: (0, 0)),
        ],
        out_specs=pl.BlockSpec((_BM, w.shape[1]), lambda i: (i, 0)),
        out_shape=jax.ShapeDtypeStruct((N, w.shape[1]), jnp.float32),
    )(x, w)


def _tc_mid(pa, pb, w):
    return pl.pallas_call(
        _mid_body,
        grid=(N // _BM,),
        in_specs=[
            pl.BlockSpec((_BM, H), lambda i: (i, 0)),
            pl.BlockSpec((_BM, H), lambda i: (i, 0)),
            pl.BlockSpec((H, H), lambda i: (0, 0)),
        ],
        out_specs=pl.BlockSpec((_BM, H), lambda i: (i, 0)),
        out_shape=jax.ShapeDtypeStruct((N, H), jnp.float32),
    )(pa, pb, w)


def _tc_head(pa, pb, eps, wm1, bm1, wm2, bm2, wl1, bl1, wl2, bl2):
    full = lambda shape: pl.BlockSpec(shape, lambda i: tuple(0 for _ in shape))
    return pl.pallas_call(
        _head_body,
        grid=(N // _BM,),
        in_specs=[
            pl.BlockSpec((_BM, H), lambda i: (i, 0)),
            pl.BlockSpec((_BM, H), lambda i: (i, 0)),
            pl.BlockSpec((_BM, OUT), lambda i: (i, 0)),
            full(wm1.shape), full((1, OUT // 2)),
            full(wm2.shape), full((1, OUT)),
            full(wl1.shape), full((1, OUT // 2)),
            full(wl2.shape), full((1, OUT)),
        ],
        out_specs=pl.BlockSpec((_BM, OUT), lambda i: (i, 0)),
        out_shape=jax.ShapeDtypeStruct((N, OUT), jnp.float32),
    )(pa, pb, eps, wm1, bm1.reshape(1, -1), wm2, bm2.reshape(1, -1),
      wl1, bl1.reshape(1, -1), wl2, bl2.reshape(1, -1))


def kernel(x, edge_index, eps, W0, W1, Wm1, bm1, Wm2, bm2, Wl1, bl1, Wl2, bl2):
    src = edge_index[0].astype(jnp.int32)
    dst = edge_index[1].astype(jnp.int32)
    pad = E_PAD - E
    src_p = jnp.concatenate([src, jnp.zeros((pad,), jnp.int32)])
    dst_p = jnp.concatenate([dst, jnp.full((pad,), N, jnp.int32)])
    src_p = src_p.reshape(NW, CH, LN)
    dst_p = dst_p.reshape(NW, CH, LN)
    zeros = jnp.zeros((N_PAD, H), jnp.float32)

    s0 = _tc_matmul(x, W0)
    p0 = _segsum_sc(s0, src_p, dst_p, zeros)
    s1 = _tc_mid(p0[0, :N], p0[1, :N], W1)
    p1 = _segsum_sc(s1, src_p, dst_p, zeros)
    return _tc_head(p1[0, :N], p1[1, :N], eps, Wm1, bm1, Wm2, bm2,
                    Wl1, bl1, Wl2, bl2)


# speedup vs baseline: 5.4481x; 1.0511x over previous
import functools

import jax
import jax.numpy as jnp
from jax import lax
from jax.experimental import pallas as pl
from jax.experimental.pallas import tpu as pltpu
from jax.experimental.pallas import tpu_sc as plsc

N = 10000
E = 320000
D_IN = 128
H = 64
OUT = 32

NW = 32
LN = 128
K0 = 128
K1 = 32
TOTCH = 16 * (K0 + K1)
BASE1 = 16 * K0
CH_PAD = TOTCH + K0 - K1
E_PAD = TOTCH * LN
N_PAD = 10112
RPT = N_PAD // 16

_mesh = plsc.VectorSubcoreMesh(core_axis_name="c", subcore_axis_name="s")


@functools.partial(
    pl.kernel,
    mesh=_mesh,
    compiler_params=pltpu.CompilerParams(use_tc_tiling_on_sc=False),
    out_type=jax.ShapeDtypeStruct((2, N_PAD, H), jnp.float32),
    scratch_types=[
        pltpu.VMEM((K0, LN), jnp.int32),
        pltpu.VMEM((K0, LN), jnp.int32),
        pltpu.VMEM((LN, H), jnp.float32),
        pltpu.VMEM_SHARED((N_PAD, H), jnp.float32),
        pltpu.SemaphoreType.DMA,
    ],
)
def _segsum_sc(rows_hbm, src_hbm, dst_hbm, zeros_hbm, out_hbm,
               src_v, dst_v, buf_v, acc_sh, sem):
    c = lax.axis_index("c")
    s = lax.axis_index("s")

    off = jnp.where(c == 0, s * K0, BASE1 + s * K1)
    nch = jnp.where(c == 0, K0, K1)
    pltpu.sync_copy(src_hbm.at[pl.ds(off, K0)], src_v)
    pltpu.sync_copy(dst_hbm.at[pl.ds(off, K0)], dst_v)
    pltpu.sync_copy(zeros_hbm.at[pl.ds(s * RPT, RPT)],
                    acc_sh.at[pl.ds(s * RPT, RPT)])
    plsc.subcore_barrier()

    def body(j, carry):
        pltpu.async_copy(rows_hbm.at[src_v.at[j]], buf_v, sem).wait()
        pltpu.sync_copy(buf_v, acc_sh.at[dst_v.at[j]], add=True)
        return carry

    lax.fori_loop(0, nch, body, 0)
    plsc.subcore_barrier()

    pltpu.sync_copy(acc_sh.at[pl.ds(s * RPT, RPT)],
                    out_hbm.at[c, pl.ds(s * RPT, RPT)])


def _mm_body(x_ref, w_ref, o_ref):
    o_ref[...] = jnp.dot(x_ref[...], w_ref[...],
                         preferred_element_type=jnp.float32)


def _mid_body(a_ref, b_ref, w_ref, o_ref):
    m = a_ref[...] + b_ref[...]
    n = jnp.sqrt(jnp.sum(m * m, axis=1, keepdims=True))
    h = jnp.maximum(m / jnp.maximum(n, 1e-12), 0.0)
    o_ref[...] = jnp.dot(h, w_ref[...], preferred_element_type=jnp.float32)


def _head_body(a_ref, b_ref, eps_ref, wm1_ref, bm1_ref, wm2_ref, bm2_ref,
               wl1_ref, bl1_ref, wl2_ref, bl2_ref, o_ref):
    m = a_ref[...] + b_ref[...]
    n = jnp.sqrt(jnp.sum(m * m, axis=1, keepdims=True))
    h = jnp.maximum(m / jnp.maximum(n, 1e-12), 0.0)
    t1 = jnp.tanh(jnp.dot(h, wm1_ref[...], preferred_element_type=jnp.float32)
                  + bm1_ref[...])
    mu = jnp.dot(t1, wm2_ref[...], preferred_element_type=jnp.float32) \
        + bm2_ref[...]
    t2 = jnp.tanh(jnp.dot(h, wl1_ref[...], preferred_element_type=jnp.float32)
                  + bl1_ref[...])
    ls = jnp.dot(t2, wl2_ref[...], preferred_element_type=jnp.float32) \
        + bl2_ref[...]
    ls = jnp.minimum(ls, 10.0)
    o_ref[...] = mu + eps_ref[...] * jnp.exp(ls)


_BM = 1000


def _tc_matmul(x, w):
    return pl.pallas_call(
        _mm_body,
        grid=(N // _BM,),
        in_specs=[
            pl.BlockSpec((_BM, x.shape[1]), lambda i: (i, 0)),
            pl.BlockSpec(w.shape, lambda i: (0, 0)),
        ],
        out_specs=pl.BlockSpec((_BM, w.shape[1]), lambda i: (i, 0)),
        out_shape=jax.ShapeDtypeStruct((N, w.shape[1]), jnp.float32),
    )(x, w)


def _tc_mid(pa, pb, w):
    return pl.pallas_call(
        _mid_body,
        grid=(N // _BM,),
        in_specs=[
            pl.BlockSpec((_BM, H), lambda i: (i, 0)),
            pl.BlockSpec((_BM, H), lambda i: (i, 0)),
            pl.BlockSpec((H, H), lambda i: (0, 0)),
        ],
        out_specs=pl.BlockSpec((_BM, H), lambda i: (i, 0)),
        out_shape=jax.ShapeDtypeStruct((N, H), jnp.float32),
    )(pa, pb, w)


def _tc_head(pa, pb, eps, wm1, bm1, wm2, bm2, wl1, bl1, wl2, bl2):
    full = lambda shape: pl.BlockSpec(shape, lambda i: tuple(0 for _ in shape))
    return pl.pallas_call(
        _head_body,
        grid=(N // _BM,),
        in_specs=[
            pl.BlockSpec((_BM, H), lambda i: (i, 0)),
            pl.BlockSpec((_BM, H), lambda i: (i, 0)),
            pl.BlockSpec((_BM, OUT), lambda i: (i, 0)),
            full(wm1.shape), full((1, OUT // 2)),
            full(wm2.shape), full((1, OUT)),
            full(wl1.shape), full((1, OUT // 2)),
            full(wl2.shape), full((1, OUT)),
        ],
        out_specs=pl.BlockSpec((_BM, OUT), lambda i: (i, 0)),
        out_shape=jax.ShapeDtypeStruct((N, OUT), jnp.float32),
    )(pa, pb, eps, wm1, bm1.reshape(1, -1), wm2, bm2.reshape(1, -1),
      wl1, bl1.reshape(1, -1), wl2, bl2.reshape(1, -1))


def kernel(x, edge_index, eps, W0, W1, Wm1, bm1, Wm2, bm2, Wl1, bl1, Wl2, bl2):
    src = edge_index[0].astype(jnp.int32)
    dst = edge_index[1].astype(jnp.int32)
    pad = CH_PAD * LN - E
    src_p = jnp.concatenate([src, jnp.zeros((pad,), jnp.int32)])
    dst_p = jnp.concatenate([dst, jnp.full((pad,), N, jnp.int32)])
    src_p = src_p.reshape(CH_PAD, LN)
    dst_p = dst_p.reshape(CH_PAD, LN)
    zeros = jnp.zeros((N_PAD, H), jnp.float32)

    s0 = _tc_matmul(x, W0)
    p0 = _segsum_sc(s0, src_p, dst_p, zeros)
    s1 = _tc_mid(p0[0, :N], p0[1, :N], W1)
    p1 = _segsum_sc(s1, src_p, dst_p, zeros)
    return _tc_head(p1[0, :N], p1[1, :N], eps, Wm1, bm1, Wm2, bm2,
                    Wl1, bl1, Wl2, bl2)
